# Initial kernel scaffold; baseline (speedup 1.0000x reference)
import jax
import jax.numpy as jnp
from jax.experimental import pallas as pl


def kernel(user_emb, item_emb, adj_row, adj_col, adj_val):
    raise NotImplementedError("write your pallas kernel here")


# speedup vs baseline: 8.6196x; 8.6196x over previous
import functools

import jax
import jax.numpy as jnp
from jax import lax
from jax.experimental import pallas as pl
from jax.experimental.pallas import tpu as pltpu
from jax.experimental.pallas import tpu_sc as plsc

N_LAYERS_ = 3
NUM_TECS = 16
BLK = 128
CHUNK_BLKS = 6
CHUNK_E = BLK * CHUNK_BLKS


def _propagate(n_pad, nnz_pad, blk_per_tec, ego0, rows2d, cols, vals):
    n2 = 2 * n_pad
    rows_per_tec = n_pad // NUM_TECS
    drain_chunk = 136
    n_drain = rows_per_tec // drain_chunk
    zrows = 184
    n_zero = rows_per_tec // zrows
    assert zrows <= CHUNK_E and 3 * drain_chunk <= CHUNK_E
    n_chunks = blk_per_tec // CHUNK_BLKS
    f32 = jnp.float32

    mesh = plsc.VectorSubcoreMesh(core_axis_name="c", subcore_axis_name="s")

    @functools.partial(
        pl.kernel,
        mesh=mesh,
        compiler_params=pltpu.CompilerParams(use_tc_tiling_on_sc=False),
        out_type=(
            jax.ShapeDtypeStruct((n2, 32), f32),
            jax.ShapeDtypeStruct((n2, 32), f32),
            jax.ShapeDtypeStruct((n2, 32), f32),
        ),
        scratch_types=[
            pltpu.VMEM_SHARED((n_pad, 32), f32),
            pltpu.VMEM((CHUNK_E, 32), f32),
            pltpu.VMEM((CHUNK_E,), jnp.int32),
            pltpu.VMEM((CHUNK_E,), f32),
            pltpu.VMEM((CHUNK_BLKS, BLK), jnp.int32),
            pltpu.SemaphoreType.DMA,
        ],
    )
    def body(ego_h, rows_h, cols_h, vals_h, out_h, l1_h, l2_h,
             acc, gbuf, colv, valv, rowv, sem):
        cid = lax.axis_index("c")
        tid = lax.axis_index("s")
        cid_off = cid * n_pad
        row0 = tid * rows_per_tec
        zeros16 = jnp.zeros((16,), f32)

        def scatter_chunk(table_h, c, _):
            blkb = tid * blk_per_tec + c * CHUNK_BLKS
            eb = blkb * BLK
            pltpu.sync_copy(rows_h.at[pl.ds(blkb, CHUNK_BLKS)], rowv)
            pltpu.sync_copy(cols_h.at[pl.ds(eb, CHUNK_E)], colv)
            pltpu.sync_copy(vals_h.at[pl.ds(eb, CHUNK_E)], valv)

            def ob(i, _):
                colv[pl.ds(i * 16, 16)] = colv[pl.ds(i * 16, 16)] + cid_off
                return _
            lax.fori_loop(0, CHUNK_E // 16, ob, None)

            cps = []
            for j in range(CHUNK_BLKS):
                cp = pltpu.make_async_copy(
                    table_h.at[colv.at[pl.ds(j * BLK, BLK)]],
                    gbuf.at[pl.ds(j * BLK, BLK)],
                    sem,
                )
                cp.start()
                cps.append(cp)
            for cp in cps:
                cp.wait()

            def mb(g, _):
                vv = valv[pl.ds(g * 16, 16)]
                base = g * 16
                for e in range(16):
                    v = vv[e]
                    gbuf[base + e, pl.ds(0, 16)] = gbuf[base + e, pl.ds(0, 16)] * v
                    gbuf[base + e, pl.ds(16, 16)] = gbuf[base + e, pl.ds(16, 16)] * v
                return _
            lax.fori_loop(0, CHUNK_E // 16, mb, None)

            for j in range(CHUNK_BLKS):
                pltpu.sync_copy(
                    gbuf.at[pl.ds(j * BLK, BLK)],
                    acc.at[rowv.at[j]],
                    add=True,
                )
            return _

        tables = (ego_h, l1_h, l2_h)
        for layer in range(N_LAYERS_):
            def zb(i, _):
                gbuf[i, pl.ds(0, 16)] = zeros16
                gbuf[i, pl.ds(16, 16)] = zeros16
                return _
            lax.fori_loop(0, zrows, zb, None)
            for z in range(n_zero):
                pltpu.sync_copy(gbuf.at[pl.ds(0, zrows)],
                                acc.at[pl.ds(row0 + z * zrows, zrows)])
            plsc.subcore_barrier()

            lax.fori_loop(
                0, n_chunks,
                functools.partial(scatter_chunk, tables[layer]),
                None,
            )
            plsc.subcore_barrier()

            if layer < N_LAYERS_ - 1:
                dst = l1_h if layer == 0 else l2_h
                pltpu.sync_copy(
                    acc.at[pl.ds(row0, rows_per_tec)],
                    dst.at[pl.ds(cid_off + row0, rows_per_tec)],
                )
            else:
                third = f32(1.0 / 3.0)

                db_o = drain_chunk
                dc_o = 2 * drain_chunk

                def dr(ch, _):
                    off = row0 + ch * drain_chunk
                    pltpu.sync_copy(acc.at[pl.ds(off, drain_chunk)],
                                    gbuf.at[pl.ds(0, drain_chunk)])
                    pltpu.sync_copy(l1_h.at[pl.ds(cid_off + off, drain_chunk)],
                                    gbuf.at[pl.ds(db_o, drain_chunk)])
                    pltpu.sync_copy(l2_h.at[pl.ds(cid_off + off, drain_chunk)],
                                    gbuf.at[pl.ds(dc_o, drain_chunk)])

                    def mr(r, _):
                        for h in (0, 16):
                            s = (gbuf[r, pl.ds(h, 16)]
                                 + gbuf[db_o + r, pl.ds(h, 16)]
                                 + gbuf[dc_o + r, pl.ds(h, 16)]) * third
                            gbuf[r, pl.ds(h, 16)] = s
                        return _
                    lax.fori_loop(0, drain_chunk, mr, None)
                    pltpu.sync_copy(gbuf.at[pl.ds(0, drain_chunk)],
                                    out_h.at[pl.ds(cid_off + off, drain_chunk)])
                    return _
                lax.fori_loop(0, n_drain, dr, None)
            plsc.subcore_barrier()

    return body(ego0, rows2d, cols, vals)


def kernel(user_emb, item_emb, adj_row, adj_col, adj_val):
    n_user = user_emb.shape[0]
    n_item = item_emb.shape[0]
    n_nodes = n_user + n_item
    n_pad = -(-n_nodes // (NUM_TECS * 8)) * (NUM_TECS * 8)
    nnz = adj_row.shape[0]

    per_tec = -(-nnz // (NUM_TECS * BLK))
    blk_per_tec = -(-per_tec // CHUNK_BLKS) * CHUNK_BLKS
    nnz_pad = blk_per_tec * NUM_TECS * BLK
    pad = nnz_pad - nnz
    rows_p = jnp.pad(adj_row, (0, pad))
    cols_p = jnp.pad(adj_col, (0, pad))
    vals_p = jnp.pad(adj_val, (0, pad))
    rows2d = rows_p.reshape(nnz_pad // BLK, BLK)

    ego = jnp.concatenate([user_emb, item_emb], axis=0)
    rpad = n_pad - n_nodes
    lo = jnp.pad(ego[:, :32], ((0, rpad), (0, 0)))
    hi = jnp.pad(ego[:, 32:], ((0, rpad), (0, 0)))
    ego0 = jnp.concatenate([lo, hi], axis=0)

    out, _l1, _l2 = _propagate(n_pad, nnz_pad, blk_per_tec,
                               ego0, rows2d, cols_p, vals_p)
    all_emb = jnp.concatenate([out[:n_nodes], out[n_pad:n_pad + n_nodes]],
                              axis=1)
    return (all_emb[:n_user], all_emb[n_user:])


# speedup vs baseline: 11.8162x; 1.3709x over previous
import functools

import jax
import jax.numpy as jnp
from jax import lax
from jax.experimental import pallas as pl
from jax.experimental.pallas import tpu as pltpu
from jax.experimental.pallas import tpu_sc as plsc

N_LAYERS_ = 3
NUM_TECS = 16
BLK = 128
CHUNK_BLKS = 3
CHUNK_E = BLK * CHUNK_BLKS
NSETS = 4


def _propagate(n_pad, nnz_pad, blk_per_tec, ego0, rows2d, cols2, vals):
    n2 = 2 * n_pad
    rows_per_tec = n_pad // NUM_TECS
    drain_chunk = 136
    n_drain = rows_per_tec // drain_chunk
    zrows = 184
    n_zero = rows_per_tec // zrows
    assert zrows <= 2 * CHUNK_E and 3 * drain_chunk <= 2 * CHUNK_E
    n_chunks = blk_per_tec // CHUNK_BLKS
    assert n_chunks % 4 == 0 and n_chunks >= 8
    n_quads = n_chunks // 4
    f32 = jnp.float32

    mesh = plsc.VectorSubcoreMesh(core_axis_name="c", subcore_axis_name="s")

    @functools.partial(
        pl.kernel,
        mesh=mesh,
        compiler_params=pltpu.CompilerParams(use_tc_tiling_on_sc=False),
        out_type=(
            jax.ShapeDtypeStruct((n2, 32), f32),
            jax.ShapeDtypeStruct((n2, 32), f32),
            jax.ShapeDtypeStruct((n2, 32), f32),
        ),
        scratch_types=[
            pltpu.VMEM_SHARED((n_pad, 32), f32),
            pltpu.VMEM((2 * CHUNK_E, 32), f32),
            pltpu.VMEM((NSETS * CHUNK_E,), jnp.int32),
            pltpu.VMEM((NSETS * CHUNK_E,), f32),
            pltpu.VMEM((NSETS * CHUNK_BLKS, BLK), jnp.int32),
            pltpu.SemaphoreType.DMA((NSETS,)),
            pltpu.SemaphoreType.DMA((2,)),
            pltpu.SemaphoreType.DMA((2,)),
        ],
    )
    def body(ego_h, rows_h, cols_h, vals_h, out_h, l1_h, l2_h,
             acc, gbuf, colv, valv, rowv, lsem, gsem, ssem):
        cid = lax.axis_index("c")
        tid = lax.axis_index("s")
        cid_off = cid * n_pad
        row0 = tid * rows_per_tec
        zeros16 = jnp.zeros((16,), f32)

        def loads_descs(s4, c):
            blkb = tid * blk_per_tec + c * CHUNK_BLKS
            eb = blkb * BLK
            return (
                pltpu.make_async_copy(
                    rows_h.at[pl.ds(blkb, CHUNK_BLKS)],
                    rowv.at[pl.ds(s4 * CHUNK_BLKS, CHUNK_BLKS)],
                    lsem.at[s4]),
                pltpu.make_async_copy(
                    cols_h.at[cid, pl.ds(eb, CHUNK_E)],
                    colv.at[pl.ds(s4 * CHUNK_E, CHUNK_E)],
                    lsem.at[s4]),
                pltpu.make_async_copy(
                    vals_h.at[pl.ds(eb, CHUNK_E)],
                    valv.at[pl.ds(s4 * CHUNK_E, CHUNK_E)],
                    lsem.at[s4]),
            )

        def gather_descs(table_h, s4, g2):
            return tuple(
                pltpu.make_async_copy(
                    table_h.at[colv.at[pl.ds(s4 * CHUNK_E + j * BLK, BLK)]],
                    gbuf.at[pl.ds(g2 * CHUNK_E + j * BLK, BLK)],
                    gsem.at[g2])
                for j in range(CHUNK_BLKS))

        def scatter_descs(s4, g2):
            return tuple(
                pltpu.make_async_copy(
                    gbuf.at[pl.ds(g2 * CHUNK_E + j * BLK, BLK)],
                    acc.at[rowv.at[s4 * CHUNK_BLKS + j]],
                    ssem.at[g2])
                for j in range(CHUNK_BLKS))

        def multiply(s4, g2):
            def mb(g, _):
                vv = valv[pl.ds(s4 * CHUNK_E + g * 16, 16)]
                base = g2 * CHUNK_E + g * 16
                for e in range(16):
                    v = vv[e]
                    gbuf[base + e, pl.ds(0, 16)] = gbuf[base + e, pl.ds(0, 16)] * v
                    gbuf[base + e, pl.ds(16, 16)] = gbuf[base + e, pl.ds(16, 16)] * v
                return _
            lax.fori_loop(0, CHUNK_E // 16, mb, None)

        def process_edges(table_h):
            for d in loads_descs(0, 0):
                d.start()
            for d in loads_descs(1, 1):
                d.start()

            def quad(q, _):
                for i in range(4):
                    c = 4 * q + i
                    g2 = i % 2
                    s4m, g2m = (i - 1) % 4, (i - 1) % 2

                    for d in loads_descs(i, c):
                        d.wait()

                    def w2(i=i, g2=g2):
                        for d in scatter_descs((i - 2) % 4, g2):
                            d.wait()
                    if i >= 2:
                        w2()
                    else:
                        pl.when(q >= 1)(w2)

                    for d in gather_descs(table_h, i, g2):
                        d.start()

                    def p4(c=c, s4m=s4m, g2m=g2m):
                        for d in gather_descs(table_h, s4m, g2m):
                            d.wait()
                        multiply(s4m, g2m)
                        for d in scatter_descs(s4m, g2m):
                            d.start(add=True)
                    if i >= 1:
                        p4()
                    else:
                        pl.when(q >= 1)(p4)

                    def l5(c=c, i=i):
                        for d in loads_descs((i + 2) % 4, c + 2):
                            d.start()
                    if i < 2:
                        l5()
                    else:
                        pl.when(q < n_quads - 1)(l5)
                return _

            lax.fori_loop(0, n_quads, quad, None)

            for d in gather_descs(table_h, 3, 1):
                d.wait()
            multiply(3, 1)
            for d in scatter_descs(3, 1):
                d.start(add=True)
            for d in scatter_descs(2, 0):
                d.wait()
            for d in scatter_descs(3, 1):
                d.wait()

        tables = (ego_h, l1_h, l2_h)
        for layer in range(N_LAYERS_):
            def zb(i, _):
                gbuf[i, pl.ds(0, 16)] = zeros16
                gbuf[i, pl.ds(16, 16)] = zeros16
                return _
            lax.fori_loop(0, zrows, zb, None)
            for z in range(n_zero):
                pltpu.sync_copy(gbuf.at[pl.ds(0, zrows)],
                                acc.at[pl.ds(row0 + z * zrows, zrows)])
            plsc.subcore_barrier()

            process_edges(tables[layer])
            plsc.subcore_barrier()

            if layer < N_LAYERS_ - 1:
                dst = l1_h if layer == 0 else l2_h
                pltpu.sync_copy(
                    acc.at[pl.ds(row0, rows_per_tec)],
                    dst.at[pl.ds(cid_off + row0, rows_per_tec)],
                )
            else:
                third = f32(1.0 / 3.0)
                db_o = drain_chunk
                dc_o = 2 * drain_chunk

                def dr(ch, _):
                    off = row0 + ch * drain_chunk
                    pltpu.sync_copy(acc.at[pl.ds(off, drain_chunk)],
                                    gbuf.at[pl.ds(0, drain_chunk)])
                    pltpu.sync_copy(l1_h.at[pl.ds(cid_off + off, drain_chunk)],
                                    gbuf.at[pl.ds(db_o, drain_chunk)])
                    pltpu.sync_copy(l2_h.at[pl.ds(cid_off + off, drain_chunk)],
                                    gbuf.at[pl.ds(dc_o, drain_chunk)])

                    def mr(r, _):
                        for h in (0, 16):
                            s = (gbuf[r, pl.ds(h, 16)]
                                 + gbuf[db_o + r, pl.ds(h, 16)]
                                 + gbuf[dc_o + r, pl.ds(h, 16)]) * third
                            gbuf[r, pl.ds(h, 16)] = s
                        return _
                    lax.fori_loop(0, drain_chunk, mr, None)
                    pltpu.sync_copy(gbuf.at[pl.ds(0, drain_chunk)],
                                    out_h.at[pl.ds(cid_off + off, drain_chunk)])
                    return _
                lax.fori_loop(0, n_drain, dr, None)
            plsc.subcore_barrier()

    return body(ego0, rows2d, cols2, vals)


def kernel(user_emb, item_emb, adj_row, adj_col, adj_val):
    n_user = user_emb.shape[0]
    n_item = item_emb.shape[0]
    n_nodes = n_user + n_item
    n_pad = -(-n_nodes // (NUM_TECS * 8)) * (NUM_TECS * 8)
    nnz = adj_row.shape[0]

    quant = CHUNK_BLKS * 4
    per_tec = -(-nnz // (NUM_TECS * BLK))
    blk_per_tec = -(-per_tec // quant) * quant
    nnz_pad = blk_per_tec * NUM_TECS * BLK
    pad = nnz_pad - nnz
    rows_p = jnp.pad(adj_row, (0, pad))
    cols_p = jnp.pad(adj_col, (0, pad))
    vals_p = jnp.pad(adj_val, (0, pad))
    rows2d = rows_p.reshape(nnz_pad // BLK, BLK)
    cols2 = jnp.stack([cols_p, cols_p + jnp.int32(n_pad)])

    ego = jnp.concatenate([user_emb, item_emb], axis=0)
    rpad = n_pad - n_nodes
    lo = jnp.pad(ego[:, :32], ((0, rpad), (0, 0)))
    hi = jnp.pad(ego[:, 32:], ((0, rpad), (0, 0)))
    ego0 = jnp.concatenate([lo, hi], axis=0)

    out, _l1, _l2 = _propagate(n_pad, nnz_pad, blk_per_tec,
                               ego0, rows2d, cols2, vals_p)
    all_emb = jnp.concatenate([out[:n_nodes], out[n_pad:n_pad + n_nodes]],
                              axis=1)
    return (all_emb[:n_user], all_emb[n_user:])


# speedup vs baseline: 11.8182x; 1.0002x over previous
import functools

import jax
import jax.numpy as jnp
from jax import lax
from jax.experimental import pallas as pl
from jax.experimental.pallas import tpu as pltpu
from jax.experimental.pallas import tpu_sc as plsc

N_LAYERS_ = 3
NUM_TECS = 16
BLK = 128
CHUNK_BLKS = 3
CHUNK_E = BLK * CHUNK_BLKS
NSETS = 4


def _propagate(n_pad, nnz_pad, blk_per_tec, ego0, rows2d, cols2, vals):
    n2 = 2 * n_pad
    rows_per_tec = n_pad // NUM_TECS
    drain_chunk = 136
    n_drain = rows_per_tec // drain_chunk
    zrows = 184
    n_zero = rows_per_tec // zrows
    assert zrows <= 2 * CHUNK_E and 3 * drain_chunk <= 2 * CHUNK_E
    n_chunks = blk_per_tec // CHUNK_BLKS
    assert n_chunks % 4 == 0 and n_chunks >= 8
    n_quads = n_chunks // 4
    f32 = jnp.float32

    mesh = plsc.VectorSubcoreMesh(core_axis_name="c", subcore_axis_name="s")

    @functools.partial(
        pl.kernel,
        mesh=mesh,
        compiler_params=pltpu.CompilerParams(use_tc_tiling_on_sc=False),
        out_type=(
            jax.ShapeDtypeStruct((n2, 32), f32),
            jax.ShapeDtypeStruct((n2, 32), f32),
            jax.ShapeDtypeStruct((n2, 32), f32),
        ),
        scratch_types=[
            pltpu.VMEM_SHARED((n_pad, 32), f32),
            pltpu.VMEM((2 * CHUNK_E, 32), f32),
            pltpu.VMEM((NSETS * CHUNK_E,), jnp.int32),
            pltpu.VMEM((NSETS * CHUNK_E,), f32),
            pltpu.VMEM((NSETS * CHUNK_BLKS, BLK), jnp.int32),
            pltpu.SemaphoreType.DMA((NSETS,)),
            pltpu.SemaphoreType.DMA((2,)),
            pltpu.SemaphoreType.DMA((2,)),
        ],
    )
    def body(ego_h, rows_h, cols_h, vals_h, out_h, l1_h, l2_h,
             acc, gbuf, colv, valv, rowv, lsem, gsem, ssem):
        cid = lax.axis_index("c")
        tid = lax.axis_index("s")
        cid_off = cid * n_pad
        row0 = tid * rows_per_tec
        zeros16 = jnp.zeros((16,), f32)

        def loads_descs(s4, c):
            blkb = tid * blk_per_tec + c * CHUNK_BLKS
            eb = blkb * BLK
            return (
                pltpu.make_async_copy(
                    rows_h.at[pl.ds(blkb, CHUNK_BLKS)],
                    rowv.at[pl.ds(s4 * CHUNK_BLKS, CHUNK_BLKS)],
                    lsem.at[s4]),
                pltpu.make_async_copy(
                    cols_h.at[cid, pl.ds(eb, CHUNK_E)],
                    colv.at[pl.ds(s4 * CHUNK_E, CHUNK_E)],
                    lsem.at[s4]),
                pltpu.make_async_copy(
                    vals_h.at[pl.ds(eb, CHUNK_E)],
                    valv.at[pl.ds(s4 * CHUNK_E, CHUNK_E)],
                    lsem.at[s4]),
            )

        def gather_descs(table_h, s4, g2):
            return tuple(
                pltpu.make_async_copy(
                    table_h.at[colv.at[pl.ds(s4 * CHUNK_E + j * BLK, BLK)]],
                    gbuf.at[pl.ds(g2 * CHUNK_E + j * BLK, BLK)],
                    gsem.at[g2])
                for j in range(CHUNK_BLKS))

        def scatter_descs(s4, g2):
            return tuple(
                pltpu.make_async_copy(
                    gbuf.at[pl.ds(g2 * CHUNK_E + j * BLK, BLK)],
                    acc.at[rowv.at[s4 * CHUNK_BLKS + j]],
                    ssem.at[g2])
                for j in range(CHUNK_BLKS))

        def multiply(s4, g2):
            def mb(g, _):
                vv = valv[pl.ds(s4 * CHUNK_E + g * 16, 16)]
                base = g2 * CHUNK_E + g * 16
                for e in range(16):
                    bvec = vv.at[jnp.full((16,), e, jnp.int32)].get(
                        mode="promise_in_bounds")
                    gbuf[base + e, pl.ds(0, 16)] = gbuf[base + e, pl.ds(0, 16)] * bvec
                    gbuf[base + e, pl.ds(16, 16)] = gbuf[base + e, pl.ds(16, 16)] * bvec
                return _
            lax.fori_loop(0, CHUNK_E // 16, mb, None)

        def process_edges(table_h):
            for d in loads_descs(0, 0):
                d.start()
            for d in loads_descs(1, 1):
                d.start()

            def quad(q, _):
                for i in range(4):
                    c = 4 * q + i
                    g2 = i % 2
                    s4m, g2m = (i - 1) % 4, (i - 1) % 2

                    for d in loads_descs(i, c):
                        d.wait()

                    def w2(i=i, g2=g2):
                        for d in scatter_descs((i - 2) % 4, g2):
                            d.wait()
                    if i >= 2:
                        w2()
                    else:
                        pl.when(q >= 1)(w2)

                    for d in gather_descs(table_h, i, g2):
                        d.start()

                    def p4(c=c, s4m=s4m, g2m=g2m):
                        for d in gather_descs(table_h, s4m, g2m):
                            d.wait()
                        multiply(s4m, g2m)
                        for d in scatter_descs(s4m, g2m):
                            d.start(add=True)
                    if i >= 1:
                        p4()
                    else:
                        pl.when(q >= 1)(p4)

                    def l5(c=c, i=i):
                        for d in loads_descs((i + 2) % 4, c + 2):
                            d.start()
                    if i < 2:
                        l5()
                    else:
                        pl.when(q < n_quads - 1)(l5)
                return _

            lax.fori_loop(0, n_quads, quad, None)

            for d in gather_descs(table_h, 3, 1):
                d.wait()
            multiply(3, 1)
            for d in scatter_descs(3, 1):
                d.start(add=True)
            for d in scatter_descs(2, 0):
                d.wait()
            for d in scatter_descs(3, 1):
                d.wait()

        tables = (ego_h, l1_h, l2_h)
        for layer in range(N_LAYERS_):
            def zb(i, _):
                gbuf[i, pl.ds(0, 16)] = zeros16
                gbuf[i, pl.ds(16, 16)] = zeros16
                return _
            lax.fori_loop(0, zrows, zb, None)
            for z in range(n_zero):
                pltpu.sync_copy(gbuf.at[pl.ds(0, zrows)],
                                acc.at[pl.ds(row0 + z * zrows, zrows)])
            plsc.subcore_barrier()

            process_edges(tables[layer])
            plsc.subcore_barrier()

            if layer < N_LAYERS_ - 1:
                dst = l1_h if layer == 0 else l2_h
                pltpu.sync_copy(
                    acc.at[pl.ds(row0, rows_per_tec)],
                    dst.at[pl.ds(cid_off + row0, rows_per_tec)],
                )
            else:
                third = f32(1.0 / 3.0)
                db_o = drain_chunk
                dc_o = 2 * drain_chunk

                def dr(ch, _):
                    off = row0 + ch * drain_chunk
                    pltpu.sync_copy(acc.at[pl.ds(off, drain_chunk)],
                                    gbuf.at[pl.ds(0, drain_chunk)])
                    pltpu.sync_copy(l1_h.at[pl.ds(cid_off + off, drain_chunk)],
                                    gbuf.at[pl.ds(db_o, drain_chunk)])
                    pltpu.sync_copy(l2_h.at[pl.ds(cid_off + off, drain_chunk)],
                                    gbuf.at[pl.ds(dc_o, drain_chunk)])

                    def mr(r, _):
                        for h in (0, 16):
                            s = (gbuf[r, pl.ds(h, 16)]
                                 + gbuf[db_o + r, pl.ds(h, 16)]
                                 + gbuf[dc_o + r, pl.ds(h, 16)]) * third
                            gbuf[r, pl.ds(h, 16)] = s
                        return _
                    lax.fori_loop(0, drain_chunk, mr, None)
                    pltpu.sync_copy(gbuf.at[pl.ds(0, drain_chunk)],
                                    out_h.at[pl.ds(cid_off + off, drain_chunk)])
                    return _
                lax.fori_loop(0, n_drain, dr, None)
            plsc.subcore_barrier()

    return body(ego0, rows2d, cols2, vals)


def kernel(user_emb, item_emb, adj_row, adj_col, adj_val):
    n_user = user_emb.shape[0]
    n_item = item_emb.shape[0]
    n_nodes = n_user + n_item
    n_pad = -(-n_nodes // (NUM_TECS * 8)) * (NUM_TECS * 8)
    nnz = adj_row.shape[0]

    quant = CHUNK_BLKS * 4
    per_tec = -(-nnz // (NUM_TECS * BLK))
    blk_per_tec = -(-per_tec // quant) * quant
    nnz_pad = blk_per_tec * NUM_TECS * BLK
    pad = nnz_pad - nnz
    rows_p = jnp.pad(adj_row, (0, pad))
    cols_p = jnp.pad(adj_col, (0, pad))
    vals_p = jnp.pad(adj_val, (0, pad))
    rows2d = rows_p.reshape(nnz_pad // BLK, BLK)
    cols2 = jnp.stack([cols_p, cols_p + jnp.int32(n_pad)])

    ego = jnp.concatenate([user_emb, item_emb], axis=0)
    rpad = n_pad - n_nodes
    lo = jnp.pad(ego[:, :32], ((0, rpad), (0, 0)))
    hi = jnp.pad(ego[:, 32:], ((0, rpad), (0, 0)))
    ego0 = jnp.concatenate([lo, hi], axis=0)

    out, _l1, _l2 = _propagate(n_pad, nnz_pad, blk_per_tec,
                               ego0, rows2d, cols2, vals_p)
    all_emb = jnp.concatenate([out[:n_nodes], out[n_pad:n_pad + n_nodes]],
                              axis=1)
    return (all_emb[:n_user], all_emb[n_user:])


# speedup vs baseline: 11.8296x; 1.0010x over previous
import functools

import jax
import jax.numpy as jnp
from jax import lax
from jax.experimental import pallas as pl
from jax.experimental.pallas import tpu as pltpu
from jax.experimental.pallas import tpu_sc as plsc

N_LAYERS_ = 3
NUM_TECS = 16
BLK = 384
CHUNK_BLKS = 1
CHUNK_E = BLK * CHUNK_BLKS
NSETS = 4


def _propagate(n_pad, nnz_pad, blk_per_tec, ego0, rows2d, cols2, vals):
    n2 = 2 * n_pad
    rows_per_tec = n_pad // NUM_TECS
    drain_chunk = 136
    n_drain = rows_per_tec // drain_chunk
    zrows = 184
    n_zero = rows_per_tec // zrows
    assert zrows <= 2 * CHUNK_E and 3 * drain_chunk <= 2 * CHUNK_E
    n_chunks = blk_per_tec // CHUNK_BLKS
    assert n_chunks % 4 == 0 and n_chunks >= 8
    n_quads = n_chunks // 4
    f32 = jnp.float32

    mesh = plsc.VectorSubcoreMesh(core_axis_name="c", subcore_axis_name="s")

    @functools.partial(
        pl.kernel,
        mesh=mesh,
        compiler_params=pltpu.CompilerParams(use_tc_tiling_on_sc=False),
        out_type=(
            jax.ShapeDtypeStruct((n2, 32), f32),
            jax.ShapeDtypeStruct((n2, 32), f32),
            jax.ShapeDtypeStruct((n2, 32), f32),
        ),
        scratch_types=[
            pltpu.VMEM_SHARED((n_pad, 32), f32),
            pltpu.VMEM((2 * CHUNK_E, 32), f32),
            pltpu.VMEM((NSETS * CHUNK_E,), jnp.int32),
            pltpu.VMEM((NSETS * CHUNK_E,), f32),
            pltpu.VMEM((NSETS * CHUNK_BLKS, BLK), jnp.int32),
            pltpu.SemaphoreType.DMA((NSETS,)),
            pltpu.SemaphoreType.DMA((2,)),
            pltpu.SemaphoreType.DMA((2,)),
        ],
    )
    def body(ego_h, rows_h, cols_h, vals_h, out_h, l1_h, l2_h,
             acc, gbuf, colv, valv, rowv, lsem, gsem, ssem):
        cid = lax.axis_index("c")
        tid = lax.axis_index("s")
        cid_off = cid * n_pad
        row0 = tid * rows_per_tec
        zeros16 = jnp.zeros((16,), f32)

        def loads_descs(s4, c):
            blkb = tid * blk_per_tec + c * CHUNK_BLKS
            eb = blkb * BLK
            return (
                pltpu.make_async_copy(
                    rows_h.at[pl.ds(blkb, CHUNK_BLKS)],
                    rowv.at[pl.ds(s4 * CHUNK_BLKS, CHUNK_BLKS)],
                    lsem.at[s4]),
                pltpu.make_async_copy(
                    cols_h.at[cid, pl.ds(eb, CHUNK_E)],
                    colv.at[pl.ds(s4 * CHUNK_E, CHUNK_E)],
                    lsem.at[s4]),
                pltpu.make_async_copy(
                    vals_h.at[pl.ds(eb, CHUNK_E)],
                    valv.at[pl.ds(s4 * CHUNK_E, CHUNK_E)],
                    lsem.at[s4]),
            )

        def gather_descs(table_h, s4, g2):
            return tuple(
                pltpu.make_async_copy(
                    table_h.at[colv.at[pl.ds(s4 * CHUNK_E + j * BLK, BLK)]],
                    gbuf.at[pl.ds(g2 * CHUNK_E + j * BLK, BLK)],
                    gsem.at[g2])
                for j in range(CHUNK_BLKS))

        def scatter_descs(s4, g2):
            return tuple(
                pltpu.make_async_copy(
                    gbuf.at[pl.ds(g2 * CHUNK_E + j * BLK, BLK)],
                    acc.at[rowv.at[s4 * CHUNK_BLKS + j]],
                    ssem.at[g2])
                for j in range(CHUNK_BLKS))

        def multiply(s4, g2):
            def mb(g, _):
                vv = valv[pl.ds(s4 * CHUNK_E + g * 16, 16)]
                base = g2 * CHUNK_E + g * 16
                for e in range(16):
                    bvec = vv.at[jnp.full((16,), e, jnp.int32)].get(
                        mode="promise_in_bounds")
                    gbuf[base + e, pl.ds(0, 16)] = gbuf[base + e, pl.ds(0, 16)] * bvec
                    gbuf[base + e, pl.ds(16, 16)] = gbuf[base + e, pl.ds(16, 16)] * bvec
                return _
            lax.fori_loop(0, CHUNK_E // 16, mb, None)

        def process_edges(table_h):
            for d in loads_descs(0, 0):
                d.start()
            for d in loads_descs(1, 1):
                d.start()

            def quad(q, _):
                for i in range(4):
                    c = 4 * q + i
                    g2 = i % 2
                    s4m, g2m = (i - 1) % 4, (i - 1) % 2

                    for d in loads_descs(i, c):
                        d.wait()

                    def w2(i=i, g2=g2):
                        for d in scatter_descs((i - 2) % 4, g2):
                            d.wait()
                    if i >= 2:
                        w2()
                    else:
                        pl.when(q >= 1)(w2)

                    for d in gather_descs(table_h, i, g2):
                        d.start()

                    def p4(c=c, s4m=s4m, g2m=g2m):
                        for d in gather_descs(table_h, s4m, g2m):
                            d.wait()
                        multiply(s4m, g2m)
                        for d in scatter_descs(s4m, g2m):
                            d.start(add=True)
                    if i >= 1:
                        p4()
                    else:
                        pl.when(q >= 1)(p4)

                    def l5(c=c, i=i):
                        for d in loads_descs((i + 2) % 4, c + 2):
                            d.start()
                    if i < 2:
                        l5()
                    else:
                        pl.when(q < n_quads - 1)(l5)
                return _

            lax.fori_loop(0, n_quads, quad, None)

            for d in gather_descs(table_h, 3, 1):
                d.wait()
            multiply(3, 1)
            for d in scatter_descs(3, 1):
                d.start(add=True)
            for d in scatter_descs(2, 0):
                d.wait()
            for d in scatter_descs(3, 1):
                d.wait()

        tables = (ego_h, l1_h, l2_h)
        for layer in range(N_LAYERS_):
            def zb(i, _):
                gbuf[i, pl.ds(0, 16)] = zeros16
                gbuf[i, pl.ds(16, 16)] = zeros16
                return _
            lax.fori_loop(0, zrows, zb, None)
            for z in range(n_zero):
                pltpu.sync_copy(gbuf.at[pl.ds(0, zrows)],
                                acc.at[pl.ds(row0 + z * zrows, zrows)])
            plsc.subcore_barrier()

            process_edges(tables[layer])
            plsc.subcore_barrier()

            if layer < N_LAYERS_ - 1:
                dst = l1_h if layer == 0 else l2_h
                pltpu.sync_copy(
                    acc.at[pl.ds(row0, rows_per_tec)],
                    dst.at[pl.ds(cid_off + row0, rows_per_tec)],
                )
            else:
                third = f32(1.0 / 3.0)
                db_o = drain_chunk
                dc_o = 2 * drain_chunk

                def dr(ch, _):
                    off = row0 + ch * drain_chunk
                    pltpu.sync_copy(acc.at[pl.ds(off, drain_chunk)],
                                    gbuf.at[pl.ds(0, drain_chunk)])
                    pltpu.sync_copy(l1_h.at[pl.ds(cid_off + off, drain_chunk)],
                                    gbuf.at[pl.ds(db_o, drain_chunk)])
                    pltpu.sync_copy(l2_h.at[pl.ds(cid_off + off, drain_chunk)],
                                    gbuf.at[pl.ds(dc_o, drain_chunk)])

                    def mr(r, _):
                        for h in (0, 16):
                            s = (gbuf[r, pl.ds(h, 16)]
                                 + gbuf[db_o + r, pl.ds(h, 16)]
                                 + gbuf[dc_o + r, pl.ds(h, 16)]) * third
                            gbuf[r, pl.ds(h, 16)] = s
                        return _
                    lax.fori_loop(0, drain_chunk, mr, None)
                    pltpu.sync_copy(gbuf.at[pl.ds(0, drain_chunk)],
                                    out_h.at[pl.ds(cid_off + off, drain_chunk)])
                    return _
                lax.fori_loop(0, n_drain, dr, None)
            plsc.subcore_barrier()

    return body(ego0, rows2d, cols2, vals)


def kernel(user_emb, item_emb, adj_row, adj_col, adj_val):
    n_user = user_emb.shape[0]
    n_item = item_emb.shape[0]
    n_nodes = n_user + n_item
    n_pad = -(-n_nodes // (NUM_TECS * 8)) * (NUM_TECS * 8)
    nnz = adj_row.shape[0]

    quant = CHUNK_BLKS * 4
    per_tec = -(-nnz // (NUM_TECS * BLK))
    blk_per_tec = -(-per_tec // quant) * quant
    nnz_pad = blk_per_tec * NUM_TECS * BLK
    pad = nnz_pad - nnz
    rows_p = jnp.pad(adj_row, (0, pad))
    cols_p = jnp.pad(adj_col, (0, pad))
    vals_p = jnp.pad(adj_val, (0, pad))
    rows2d = rows_p.reshape(nnz_pad // BLK, BLK)
    cols2 = jnp.stack([cols_p, cols_p + jnp.int32(n_pad)])

    ego = jnp.concatenate([user_emb, item_emb], axis=0)
    rpad = n_pad - n_nodes
    lo = jnp.pad(ego[:, :32], ((0, rpad), (0, 0)))
    hi = jnp.pad(ego[:, 32:], ((0, rpad), (0, 0)))
    ego0 = jnp.concatenate([lo, hi], axis=0)

    out, _l1, _l2 = _propagate(n_pad, nnz_pad, blk_per_tec,
                               ego0, rows2d, cols2, vals_p)
    all_emb = jnp.concatenate([out[:n_nodes], out[n_pad:n_pad + n_nodes]],
                              axis=1)
    return (all_emb[:n_user], all_emb[n_user:])
